# Initial kernel scaffold; baseline (speedup 1.0000x reference)
import jax
import jax.numpy as jnp
from jax.experimental import pallas as pl


def kernel(x, edge_index, W1, b1, W2, b2):
    raise NotImplementedError("write your pallas kernel here")


# speedup vs baseline: 13.5161x; 13.5161x over previous
import functools

import jax
import jax.numpy as jnp
from jax import lax
from jax.experimental import pallas as pl
from jax.experimental.pallas import tpu as pltpu
from jax.experimental.pallas import tpu_sc as plsc

NC = 2
NS = 16
LANES = 16


def _sc_mesh():
    return plsc.VectorSubcoreMesh(
        core_axis_name="c", subcore_axis_name="s", num_cores=NC, num_subcores=NS
    )


def _pick_chunk(per_worker):
    for ch in range(128, 0, -8):
        if per_worker % ch == 0:
            return ch
    raise ValueError(f"no valid chunk for {per_worker}")


def _deg_histogram(col, n):
    e = col.shape[0]
    npad = -(-n // (NS * LANES)) * (NS * LANES)
    slab = npad // NS
    ew = e // (NC * NS)
    ch = _pick_chunk(ew)
    iters = ew // ch

    @functools.partial(
        pl.kernel,
        out_type=jax.ShapeDtypeStruct((NC, npad), jnp.float32),
        mesh=_sc_mesh(),
        scratch_types=[
            pltpu.VMEM((ch,), jnp.int32),
            pltpu.VMEM((ch,), jnp.float32),
            pltpu.VMEM((slab,), jnp.float32),
            pltpu.VMEM_SHARED((npad,), jnp.float32),
        ],
    )
    def k(col_hbm, out_hbm, cidx, ones, zbuf, acc):
        c = lax.axis_index("c")
        s = lax.axis_index("s")
        w = c * NS + s

        def fill(i, _):
            zbuf[pl.ds(i * LANES, LANES)] = jnp.zeros((LANES,), jnp.float32)
            return _

        lax.fori_loop(0, slab // LANES, fill, 0)
        for j in range(ch // LANES):
            ones[pl.ds(j * LANES, LANES)] = jnp.ones((LANES,), jnp.float32)
        pltpu.sync_copy(zbuf, acc.at[pl.ds(s * slab, slab)])
        plsc.subcore_barrier()

        base = w * ew

        def body(i, _):
            pltpu.sync_copy(col_hbm.at[pl.ds(base + i * ch, ch)], cidx)
            pltpu.sync_copy(ones, acc.at[cidx], add=True)
            return _

        lax.fori_loop(0, iters, body, 0)
        plsc.subcore_barrier()
        pltpu.sync_copy(acc.at[pl.ds(s * slab, slab)],
                        out_hbm.at[c, pl.ds(s * slab, slab)])

    return k(col)


def _edge_scatter(hs, row, col):
    n, d = hs.shape
    e = row.shape[0]
    npad = -(-n // (NS * 128)) * (NS * 128)
    slab = npad // NS
    ew = e // (NC * NS)
    ch = _pick_chunk(ew)
    iters = ew // ch
    zr = 8
    for cand in range(slab, 0, -8):
        if slab % cand == 0 and cand * d * 4 <= 65536:
            zr = cand
            break

    @functools.partial(
        pl.kernel,
        out_type=jax.ShapeDtypeStruct((NC, npad, d), jnp.float32),
        mesh=_sc_mesh(),
        scratch_types=[
            pltpu.VMEM((ch,), jnp.int32),
            pltpu.VMEM((ch,), jnp.int32),
            pltpu.VMEM((ch, d), jnp.float32),
            pltpu.VMEM((zr, d), jnp.float32),
            pltpu.VMEM_SHARED((npad, d), jnp.float32),
            pltpu.SemaphoreType.DMA,
        ],
    )
    def k(hs_hbm, row_hbm, col_hbm, out_hbm, ridx, cidx, rows, zbuf, acc, sem):
        c = lax.axis_index("c")
        s = lax.axis_index("s")
        w = c * NS + s

        def fill(i, _):
            for j in range(d // LANES):
                zbuf[i, pl.ds(j * LANES, LANES)] = jnp.zeros((LANES,), jnp.float32)
            return _

        lax.fori_loop(0, zr, fill, 0)
        for t in range(slab // zr):
            pltpu.sync_copy(zbuf, acc.at[pl.ds(s * slab + t * zr, zr)])
        plsc.subcore_barrier()

        base = w * ew

        def body(i, _):
            off = base + i * ch
            pltpu.sync_copy(row_hbm.at[pl.ds(off, ch)], ridx)
            pltpu.sync_copy(col_hbm.at[pl.ds(off, ch)], cidx)
            pltpu.async_copy(hs_hbm.at[ridx], rows, sem).wait()
            pltpu.sync_copy(rows, acc.at[cidx], add=True)
            return _

        lax.fori_loop(0, iters, body, 0)
        plsc.subcore_barrier()
        pltpu.sync_copy(acc.at[pl.ds(s * slab, slab)],
                        out_hbm.at[c, pl.ds(s * slab, slab)])

    return k(hs, row, col)


def _mm_pre(x, W, degp):
    n, d = x.shape

    def body(x_ref, w_ref, deg_ref, o_ref, dis_ref):
        deg = deg_ref[0, :n] + deg_ref[1, :n] + 1.0
        dis = lax.rsqrt(deg)[:, None]
        h = jnp.dot(x_ref[...], w_ref[...], preferred_element_type=jnp.float32)
        o_ref[...] = h * dis
        dis_ref[...] = dis

    return pl.pallas_call(
        body,
        out_shape=[
            jax.ShapeDtypeStruct((n, d), jnp.float32),
            jax.ShapeDtypeStruct((n, 1), jnp.float32),
        ],
    )(x, W, degp)


def _mm_mid(s1, hs1, dis, W2, b1):
    n, d = hs1.shape

    def body(s1_ref, hs1_ref, dis_ref, w_ref, b_ref, o_ref):
        dis_v = dis_ref[...]
        pre = (dis_v * (s1_ref[0, :n] + s1_ref[1, :n] + hs1_ref[...])
               + b_ref[...][None, :])
        a = jnp.maximum(pre, 0.0)
        o_ref[...] = jnp.dot(a, w_ref[...], preferred_element_type=jnp.float32) * dis_v

    return pl.pallas_call(
        body, out_shape=jax.ShapeDtypeStruct((n, d), jnp.float32)
    )(s1, hs1, dis, W2, b1)


def _mm_post(s2, hs2, dis, b2):
    n, d = hs2.shape

    def body(s2_ref, hs2_ref, dis_ref, b_ref, o_ref):
        o_ref[...] = (dis_ref[...] * (s2_ref[0, :n] + s2_ref[1, :n] + hs2_ref[...])
                      + b_ref[...][None, :])

    return pl.pallas_call(
        body, out_shape=jax.ShapeDtypeStruct((n, d), jnp.float32)
    )(s2, hs2, dis, b2)


def kernel(x, edge_index, W1, b1, W2, b2):
    n = x.shape[0]
    row = edge_index[0]
    col = edge_index[1]
    degp = _deg_histogram(col, n)
    hs1, dis = _mm_pre(x, W1, degp)
    s1 = _edge_scatter(hs1, row, col)
    hs2 = _mm_mid(s1, hs1, dis, W2, b1)
    s2 = _edge_scatter(hs2, row, col)
    return _mm_post(s2, hs2, dis, b2)


# speedup vs baseline: 25.9373x; 1.9190x over previous
import functools

import jax
import jax.numpy as jnp
from jax import lax
from jax.experimental import pallas as pl
from jax.experimental.pallas import tpu as pltpu
from jax.experimental.pallas import tpu_sc as plsc

NC = 2
NS = 16
LANES = 16


def _sc_mesh():
    return plsc.VectorSubcoreMesh(
        core_axis_name="c", subcore_axis_name="s", num_cores=NC, num_subcores=NS
    )


def _pick_chunk(per_worker):
    for ch in range(128, 0, -8):
        if per_worker % ch == 0:
            return ch
    raise ValueError(f"no valid chunk for {per_worker}")


def _deg_histogram(col, n):
    e = col.shape[0]
    npad = -(-n // (NS * LANES)) * (NS * LANES)
    slab = npad // NS
    ew = e // (NC * NS)
    ch = _pick_chunk(ew)
    k_sub = 5 if (ew // ch) % 5 == 0 else 1
    iters = ew // (ch * k_sub)

    @functools.partial(
        pl.kernel,
        out_type=jax.ShapeDtypeStruct((NC, npad), jnp.float32),
        mesh=_sc_mesh(),
        scratch_types=[
            pltpu.VMEM((k_sub, ch), jnp.int32),
            pltpu.VMEM((ch,), jnp.float32),
            pltpu.VMEM((slab,), jnp.float32),
            pltpu.VMEM_SHARED((npad,), jnp.float32),
            pltpu.SemaphoreType.DMA,
            pltpu.SemaphoreType.DMA,
        ],
    )
    def k(col_hbm, out_hbm, cidx, ones, zbuf, acc, sem_i, sem_s):
        c = lax.axis_index("c")
        s = lax.axis_index("s")
        w = c * NS + s

        def fill(i, _):
            zbuf[pl.ds(i * LANES, LANES)] = jnp.zeros((LANES,), jnp.float32)
            return _

        lax.fori_loop(0, slab // LANES, fill, 0)
        for j in range(ch // LANES):
            ones[pl.ds(j * LANES, LANES)] = jnp.ones((LANES,), jnp.float32)
        pltpu.sync_copy(zbuf, acc.at[pl.ds(s * slab, slab)])
        plsc.subcore_barrier()

        base = w * ew

        def body(i, _):
            off = base + i * (ch * k_sub)
            for j in range(k_sub):
                pltpu.async_copy(col_hbm.at[pl.ds(off + j * ch, ch)],
                                 cidx.at[j], sem_i)
            for j in range(k_sub):
                pltpu.make_async_copy(col_hbm.at[pl.ds(off + j * ch, ch)],
                                      cidx.at[j], sem_i).wait()
            for j in range(k_sub):
                pltpu.async_copy(ones, acc.at[cidx.at[j]], sem_s, add=True)
            for j in range(k_sub):
                pltpu.make_async_copy(ones, acc.at[cidx.at[j]], sem_s).wait()
            return _

        lax.fori_loop(0, iters, body, 0)
        plsc.subcore_barrier()
        pltpu.sync_copy(acc.at[pl.ds(s * slab, slab)],
                        out_hbm.at[c, pl.ds(s * slab, slab)])

    return k(col)


def _edge_scatter(hs, row, col):
    n, d = hs.shape
    e = row.shape[0]
    npad = -(-n // (NS * 128)) * (NS * 128)
    slab = npad // NS
    ew = e // (NC * NS)
    sub = _pick_chunk(ew)
    ksub = 1
    sup = sub * ksub
    g_total = ew // sup
    assert g_total % 2 == 1 and g_total >= 3, (ew, sub, ksub)
    zr = 8
    for cand in range(min(slab, 64), 0, -8):
        if slab % cand == 0:
            zr = cand
            break

    @functools.partial(
        pl.kernel,
        out_type=jax.ShapeDtypeStruct((NC, npad, d), jnp.float32),
        mesh=_sc_mesh(),
        scratch_types=[
            pltpu.VMEM((2, ksub, sub), jnp.int32),
            pltpu.VMEM((2, ksub, sub), jnp.int32),
            pltpu.VMEM((2, sup, d), jnp.float32),
            pltpu.VMEM((zr, d), jnp.float32),
            pltpu.VMEM_SHARED((npad, d), jnp.float32),
            pltpu.SemaphoreType.DMA,
            pltpu.SemaphoreType.DMA,
            pltpu.SemaphoreType.DMA,
        ],
    )
    def k(hs_hbm, row_hbm, col_hbm, out_hbm, ridx, cidx, rows, zbuf, acc,
          sem_i, sem_g, sem_s):
        c = lax.axis_index("c")
        s = lax.axis_index("s")
        w = c * NS + s
        base = w * ew

        def fire_idx(g, b):
            off = base + g * sup
            for j in range(ksub):
                pltpu.async_copy(row_hbm.at[pl.ds(off + j * sub, sub)],
                                 ridx.at[b, j], sem_i)
                pltpu.async_copy(col_hbm.at[pl.ds(off + j * sub, sub)],
                                 cidx.at[b, j], sem_i)

        def drain_idx(b):
            for j in range(ksub):
                pltpu.make_async_copy(row_hbm.at[pl.ds(base, sub)],
                                      ridx.at[b, j], sem_i).wait()
                pltpu.make_async_copy(col_hbm.at[pl.ds(base, sub)],
                                      cidx.at[b, j], sem_i).wait()

        def fire_gather(b):
            for j in range(ksub):
                pltpu.async_copy(hs_hbm.at[ridx.at[b, j]],
                                 rows.at[b, pl.ds(j * sub, sub)], sem_g)

        def drain_gather(b):
            for j in range(ksub):
                pltpu.make_async_copy(hs_hbm.at[ridx.at[b, j]],
                                      rows.at[b, pl.ds(j * sub, sub)],
                                      sem_g).wait()

        def fire_scat(b):
            for j in range(ksub):
                pltpu.async_copy(rows.at[b, pl.ds(j * sub, sub)],
                                 acc.at[cidx.at[b, j]], sem_s, add=True)

        def drain_scat(b):
            for j in range(ksub):
                pltpu.make_async_copy(rows.at[b, pl.ds(j * sub, sub)],
                                      acc.at[cidx.at[b, j]], sem_s).wait()

        def fill(i, _):
            for j in range(d // LANES):
                zbuf[i, pl.ds(j * LANES, LANES)] = jnp.zeros((LANES,), jnp.float32)
            return _

        lax.fori_loop(0, zr, fill, 0)
        for t in range(slab // zr):
            pltpu.sync_copy(zbuf, acc.at[pl.ds(s * slab + t * zr, zr)])
        plsc.subcore_barrier()

        fire_idx(0, 0)
        drain_idx(0)
        fire_gather(0)

        def pair(t, _):
            g1 = 1 + 2 * t
            g2 = 2 + 2 * t
            fire_idx(g1, 1)
            drain_gather(0)
            drain_idx(1)
            fire_gather(1)
            fire_scat(0)
            drain_scat(0)
            fire_idx(g2, 0)
            drain_gather(1)
            drain_idx(0)
            fire_gather(0)
            fire_scat(1)
            drain_scat(1)
            return _

        lax.fori_loop(0, (g_total - 1) // 2, pair, 0)
        drain_gather(0)
        fire_scat(0)
        drain_scat(0)

        plsc.subcore_barrier()
        pltpu.sync_copy(acc.at[pl.ds(s * slab, slab)],
                        out_hbm.at[c, pl.ds(s * slab, slab)])

    return k(hs, row, col)


def _mm_pre(x, W, degp):
    n, d = x.shape

    def body(x_ref, w_ref, deg_ref, o_ref, dis_ref):
        deg = deg_ref[0, :n] + deg_ref[1, :n] + 1.0
        dis = lax.rsqrt(deg)[:, None]
        h = jnp.dot(x_ref[...], w_ref[...], preferred_element_type=jnp.float32)
        o_ref[...] = h * dis
        dis_ref[...] = dis

    return pl.pallas_call(
        body,
        out_shape=[
            jax.ShapeDtypeStruct((n, d), jnp.float32),
            jax.ShapeDtypeStruct((n, 1), jnp.float32),
        ],
    )(x, W, degp)


def _mm_mid(s1, hs1, dis, W2, b1):
    n, d = hs1.shape

    def body(s1_ref, hs1_ref, dis_ref, w_ref, b_ref, o_ref):
        dis_v = dis_ref[...]
        pre = (dis_v * (s1_ref[0, :n] + s1_ref[1, :n] + hs1_ref[...])
               + b_ref[...][None, :])
        a = jnp.maximum(pre, 0.0)
        o_ref[...] = jnp.dot(a, w_ref[...], preferred_element_type=jnp.float32) * dis_v

    return pl.pallas_call(
        body, out_shape=jax.ShapeDtypeStruct((n, d), jnp.float32)
    )(s1, hs1, dis, W2, b1)


def _mm_post(s2, hs2, dis, b2):
    n, d = hs2.shape

    def body(s2_ref, hs2_ref, dis_ref, b_ref, o_ref):
        o_ref[...] = (dis_ref[...] * (s2_ref[0, :n] + s2_ref[1, :n] + hs2_ref[...])
                      + b_ref[...][None, :])

    return pl.pallas_call(
        body, out_shape=jax.ShapeDtypeStruct((n, d), jnp.float32)
    )(s2, hs2, dis, b2)


def kernel(x, edge_index, W1, b1, W2, b2):
    n = x.shape[0]
    row = edge_index[0]
    col = edge_index[1]
    degp = _deg_histogram(col, n)
    hs1, dis = _mm_pre(x, W1, degp)
    s1 = _edge_scatter(hs1, row, col)
    hs2 = _mm_mid(s1, hs1, dis, W2, b1)
    s2 = _edge_scatter(hs2, row, col)
    return _mm_post(s2, hs2, dis, b2)


# speedup vs baseline: 37.4644x; 1.4444x over previous
import functools

import jax
import jax.numpy as jnp
from jax import lax
from jax.experimental import pallas as pl
from jax.experimental.pallas import tpu as pltpu
from jax.experimental.pallas import tpu_sc as plsc

NC = 2
NS = 16
LANES = 16


def _sc_mesh():
    return plsc.VectorSubcoreMesh(
        core_axis_name="c", subcore_axis_name="s", num_cores=NC, num_subcores=NS
    )


def _pick_chunk(per_worker):
    for ch in range(128, 0, -8):
        if per_worker % ch == 0:
            return ch
    raise ValueError(f"no valid chunk for {per_worker}")


def _deg_histogram(col, n):
    e = col.shape[0]
    npad = -(-n // (NS * LANES)) * (NS * LANES)
    slab = npad // NS
    ew = e // (NC * NS)
    ch = _pick_chunk(ew)
    k_sub = 5 if (ew // ch) % 5 == 0 else 1
    iters = ew // (ch * k_sub)

    @functools.partial(
        pl.kernel,
        out_type=jax.ShapeDtypeStruct((NC, npad), jnp.float32),
        mesh=_sc_mesh(),
        scratch_types=[
            pltpu.VMEM((k_sub, ch), jnp.int32),
            pltpu.VMEM((ch,), jnp.float32),
            pltpu.VMEM((slab,), jnp.float32),
            pltpu.VMEM_SHARED((npad,), jnp.float32),
            pltpu.SemaphoreType.DMA,
            pltpu.SemaphoreType.DMA,
        ],
    )
    def k(col_hbm, out_hbm, cidx, ones, zbuf, acc, sem_i, sem_s):
        c = lax.axis_index("c")
        s = lax.axis_index("s")
        w = c * NS + s

        def fill(i, _):
            zbuf[pl.ds(i * LANES, LANES)] = jnp.zeros((LANES,), jnp.float32)
            return _

        lax.fori_loop(0, slab // LANES, fill, 0)
        for j in range(ch // LANES):
            ones[pl.ds(j * LANES, LANES)] = jnp.ones((LANES,), jnp.float32)
        pltpu.sync_copy(zbuf, acc.at[pl.ds(s * slab, slab)])
        plsc.subcore_barrier()

        base = w * ew

        def body(i, _):
            off = base + i * (ch * k_sub)
            for j in range(k_sub):
                pltpu.async_copy(col_hbm.at[pl.ds(off + j * ch, ch)],
                                 cidx.at[j], sem_i)
            for j in range(k_sub):
                pltpu.make_async_copy(col_hbm.at[pl.ds(off + j * ch, ch)],
                                      cidx.at[j], sem_i).wait()
            for j in range(k_sub):
                pltpu.async_copy(ones, acc.at[cidx.at[j]], sem_s, add=True)
            for j in range(k_sub):
                pltpu.make_async_copy(ones, acc.at[cidx.at[j]], sem_s).wait()
            return _

        lax.fori_loop(0, iters, body, 0)
        plsc.subcore_barrier()
        pltpu.sync_copy(acc.at[pl.ds(s * slab, slab)],
                        out_hbm.at[c, pl.ds(s * slab, slab)])

    return k(col)


def _edge_scatter(hs, row, col):
    n, d = hs.shape
    e = row.shape[0]
    npad = -(-n // (NS * 128)) * (NS * 128)
    slab = npad // NS
    ew = e // (NC * NS)
    sub = _pick_chunk(ew)
    g_total = ew // sub
    t_body = (g_total - 3) // 6
    r_tail = g_total - 6 * t_body
    assert g_total >= 6, (ew, sub)
    zr = 8
    for cand in range(min(slab, 64), 0, -8):
        if slab % cand == 0:
            zr = cand
            break

    @functools.partial(
        pl.kernel,
        out_type=jax.ShapeDtypeStruct((NC, npad, d), jnp.float32),
        mesh=_sc_mesh(),
        scratch_types=[
            pltpu.VMEM((6, 2, sub), jnp.int32),
            pltpu.VMEM((3, sub, d), jnp.float32),
            pltpu.VMEM((zr, d), jnp.float32),
            pltpu.VMEM_SHARED((npad, d), jnp.float32),
            pltpu.SemaphoreType.DMA,
            pltpu.SemaphoreType.DMA,
            pltpu.SemaphoreType.DMA,
            pltpu.SemaphoreType.DMA,
            pltpu.SemaphoreType.DMA,
        ],
    )
    def k(hs_hbm, row_hbm, col_hbm, out_hbm, eidx, rows, zbuf, acc,
          sem_i, sem_g0, sem_g1, sem_g2, sem_s):
        c = lax.axis_index("c")
        s = lax.axis_index("s")
        w = c * NS + s
        base = w * ew
        sem_g = (sem_g0, sem_g1, sem_g2)

        def fire_eidx(g, ib):
            off = base + g * sub
            pltpu.async_copy(row_hbm.at[pl.ds(off, sub)], eidx.at[ib, 0], sem_i)
            pltpu.async_copy(col_hbm.at[pl.ds(off, sub)], eidx.at[ib, 1], sem_i)

        def drain_eidx(ib):
            pltpu.make_async_copy(row_hbm.at[pl.ds(base, sub)],
                                  eidx.at[ib, 0], sem_i).wait()
            pltpu.make_async_copy(col_hbm.at[pl.ds(base, sub)],
                                  eidx.at[ib, 1], sem_i).wait()

        def fire_gather(rb, ib):
            pltpu.async_copy(hs_hbm.at[eidx.at[ib, 0]], rows.at[rb], sem_g[rb])

        def drain_gather(rb, ib):
            pltpu.make_async_copy(hs_hbm.at[eidx.at[ib, 0]], rows.at[rb],
                                  sem_g[rb]).wait()

        def fire_scat(rb, ib):
            pltpu.async_copy(rows.at[rb], acc.at[eidx.at[ib, 1]], sem_s,
                             add=True)

        def drain_scat(rb, ib):
            pltpu.make_async_copy(rows.at[rb], acc.at[eidx.at[ib, 1]],
                                  sem_s).wait()

        def fill(i, _):
            for j in range(d // LANES):
                zbuf[i, pl.ds(j * LANES, LANES)] = jnp.zeros((LANES,), jnp.float32)
            return _

        lax.fori_loop(0, zr, fill, 0)
        for t in range(slab // zr):
            pltpu.sync_copy(zbuf, acc.at[pl.ds(s * slab + t * zr, zr)])
        plsc.subcore_barrier()

        for g in range(3):
            fire_eidx(g, g)
        for g in range(3):
            drain_eidx(g)
        for g in range(3):
            fire_gather(g, g)

        def slot(g, u):
            rb = u % 3
            ib = u % 6
            ib2 = (u + 3) % 6
            fire_eidx(g + 3, ib2)
            drain_gather(rb, ib)
            fire_scat(rb, ib)
            drain_scat(rb, ib)
            drain_eidx(ib2)
            fire_gather(rb, ib2)

        def body(t, _):
            g0 = 6 * t
            for u in range(6):
                slot(g0 + u, u)
            return _

        lax.fori_loop(0, t_body, body, 0)

        g0 = 6 * t_body
        for u in range(r_tail):
            rb = u % 3
            ib = u % 6
            if u + 3 < r_tail:
                fire_eidx(g0 + u + 3, (u + 3) % 6)
            drain_gather(rb, ib)
            fire_scat(rb, ib)
            drain_scat(rb, ib)
            if u + 3 < r_tail:
                drain_eidx((u + 3) % 6)
                fire_gather(rb, (u + 3) % 6)

        plsc.subcore_barrier()
        pltpu.sync_copy(acc.at[pl.ds(s * slab, slab)],
                        out_hbm.at[c, pl.ds(s * slab, slab)])

    return k(hs, row, col)


def _mm_pre(x, W, degp):
    n, d = x.shape

    def body(x_ref, w_ref, deg_ref, o_ref, dis_ref):
        deg = deg_ref[0, :n] + deg_ref[1, :n] + 1.0
        dis = lax.rsqrt(deg)[:, None]
        h = jnp.dot(x_ref[...], w_ref[...], preferred_element_type=jnp.float32)
        o_ref[...] = h * dis
        dis_ref[...] = dis

    return pl.pallas_call(
        body,
        out_shape=[
            jax.ShapeDtypeStruct((n, d), jnp.float32),
            jax.ShapeDtypeStruct((n, 1), jnp.float32),
        ],
    )(x, W, degp)


def _mm_mid(s1, hs1, dis, W2, b1):
    n, d = hs1.shape

    def body(s1_ref, hs1_ref, dis_ref, w_ref, b_ref, o_ref):
        dis_v = dis_ref[...]
        pre = (dis_v * (s1_ref[0, :n] + s1_ref[1, :n] + hs1_ref[...])
               + b_ref[...][None, :])
        a = jnp.maximum(pre, 0.0)
        o_ref[...] = jnp.dot(a, w_ref[...], preferred_element_type=jnp.float32) * dis_v

    return pl.pallas_call(
        body, out_shape=jax.ShapeDtypeStruct((n, d), jnp.float32)
    )(s1, hs1, dis, W2, b1)


def _mm_post(s2, hs2, dis, b2):
    n, d = hs2.shape

    def body(s2_ref, hs2_ref, dis_ref, b_ref, o_ref):
        o_ref[...] = (dis_ref[...] * (s2_ref[0, :n] + s2_ref[1, :n] + hs2_ref[...])
                      + b_ref[...][None, :])

    return pl.pallas_call(
        body, out_shape=jax.ShapeDtypeStruct((n, d), jnp.float32)
    )(s2, hs2, dis, b2)


def kernel(x, edge_index, W1, b1, W2, b2):
    n = x.shape[0]
    row = edge_index[0]
    col = edge_index[1]
    degp = _deg_histogram(col, n)
    hs1, dis = _mm_pre(x, W1, degp)
    s1 = _edge_scatter(hs1, row, col)
    hs2 = _mm_mid(s1, hs1, dis, W2, b1)
    s2 = _edge_scatter(hs2, row, col)
    return _mm_post(s2, hs2, dis, b2)
